# Initial kernel scaffold; baseline (speedup 1.0000x reference)
import jax
import jax.numpy as jnp
from jax.experimental import pallas as pl


def kernel(x, edge_index, batch, params):
    raise NotImplementedError("write your pallas kernel here")


# speedup vs baseline: 8.1324x; 8.1324x over previous
import functools

import jax
import jax.numpy as jnp
from jax import lax
from jax.experimental import pallas as pl
from jax.experimental.pallas import tpu as pltpu
from jax.experimental.pallas import tpu_sc as plsc

N = 50000
BG = 256
NSC = 2
NTILE = 16
NWORK = NSC * NTILE
IB = 128
RB = 2048
NN = 51200
NRB = NN // RB
NP = NN + 512
RPT = NP // NTILE
NZ = 4
ZR = RPT // NZ
WB = RPT // IB + 1
EPS = 1e-5
GBK = 4

_scparams = pltpu.CompilerParams(use_tc_tiling_on_sc=False)


def _sc_mesh():
  return plsc.VectorSubcoreMesh(core_axis_name="c", subcore_axis_name="s")


def _make_prop(C, nb, mult=8):
  @functools.partial(
      pl.kernel,
      out_type=jax.ShapeDtypeStruct((NP * mult, 16), jnp.float32),
      mesh=_sc_mesh(),
      compiler_params=_scparams,
      scratch_types=[
          pltpu.VMEM((GBK, IB), jnp.int32),
          pltpu.VMEM((GBK, IB), jnp.int32),
          [pltpu.VMEM((IB, 16), jnp.float32) for _ in range(GBK)],
          pltpu.VMEM((ZR, 16), jnp.float32),
          pltpu.VMEM((WB, IB), jnp.int32),
          pltpu.VMEM((IB, 16), jnp.float32),
          pltpu.VMEM_SHARED((NP, 16), jnp.float32),
          [pltpu.SemaphoreType.DMA for _ in range(GBK)],
          pltpu.SemaphoreType.DMA,
      ],
  )
  def prop(u_hbm, gidx_hbm, didx_hbm, widx_hbm, out_hbm,
           idxg, idxd, rows, zbuf, widx, stage, acc, sems, wsem):
    cid = lax.axis_index("c")
    sid = lax.axis_index("s")

    def zfill(i, _):
      zbuf[i, :] = jnp.zeros((16,), jnp.float32)
      return 0
    lax.fori_loop(0, ZR, zfill, 0)

    for c in range(C):
      my = c % NSC

      @pl.when(cid == my)
      def _():
        for z in range(NZ):
          pltpu.sync_copy(zbuf, acc.at[pl.ds(sid * RPT + z * ZR, ZR)])
        plsc.subcore_barrier()

        def grp(g, _):
          pltpu.sync_copy(gidx_hbm.at[c, sid, pl.ds(g * GBK, GBK)], idxg)
          pltpu.sync_copy(didx_hbm.at[sid, pl.ds(g * GBK, GBK)], idxd)
          for j in range(GBK):
            pltpu.async_copy(u_hbm.at[idxg.at[j]], rows[j], sems[j])
          for j in range(GBK):
            pltpu.make_async_copy(u_hbm.at[idxg.at[j]], rows[j],
                                  sems[j]).wait()
            pltpu.sync_copy(rows[j], acc.at[idxd.at[j]], add=True)
          return 0
        lax.fori_loop(0, nb // GBK, grp, 0)
        plsc.subcore_barrier()

        pltpu.sync_copy(widx_hbm.at[c, sid], widx)

        def wb(b, _):
          pltpu.sync_copy(acc.at[pl.ds(sid * RPT + b * IB, IB)], stage)
          pltpu.async_copy(stage, out_hbm.at[widx.at[b]], wsem).wait()
          return 0
        lax.fori_loop(0, WB - 1, wb, 0)
        pltpu.sync_copy(acc.at[pl.ds(sid * RPT + RPT - IB, IB)], stage)
        pltpu.async_copy(stage, out_hbm.at[widx.at[WB - 1]], wsem).wait()

  return prop


def _make_deg(nbd):
  @functools.partial(
      pl.kernel,
      out_type=jax.ShapeDtypeStruct((NP * 8, 16), jnp.float32),
      mesh=_sc_mesh(),
      compiler_params=_scparams,
      scratch_types=[
          pltpu.VMEM((GBK, IB), jnp.int32),
          pltpu.VMEM((IB, 16), jnp.float32),
          pltpu.VMEM((ZR, 16), jnp.float32),
          pltpu.VMEM((WB, IB), jnp.int32),
          pltpu.VMEM((IB, 16), jnp.float32),
          pltpu.VMEM_SHARED((NP, 16), jnp.float32),
          pltpu.SemaphoreType.DMA,
      ],
  )
  def deg(didx_hbm, widx_hbm, out_hbm,
          idxd, ones_v, zbuf, widx, stage, acc, wsem):
    cid = lax.axis_index("c")
    sid = lax.axis_index("s")
    wid = cid * NTILE + sid

    def ofill(i, _):
      ones_v[i, :] = jnp.ones((16,), jnp.float32)
      return 0
    lax.fori_loop(0, IB, ofill, 0)

    def zfill(i, _):
      zbuf[i, :] = jnp.zeros((16,), jnp.float32)
      return 0
    lax.fori_loop(0, ZR, zfill, 0)

    for z in range(NZ):
      pltpu.sync_copy(zbuf, acc.at[pl.ds(sid * RPT + z * ZR, ZR)])
    plsc.subcore_barrier()

    def grp(g, _):
      pltpu.sync_copy(didx_hbm.at[wid, pl.ds(g * GBK, GBK)], idxd)
      for j in range(GBK):
        pltpu.sync_copy(ones_v, acc.at[idxd.at[j]], add=True)
      return 0
    lax.fori_loop(0, nbd // GBK, grp, 0)
    plsc.subcore_barrier()

    pltpu.sync_copy(widx_hbm.at[cid, sid], widx)

    def wb(b, _):
      pltpu.sync_copy(acc.at[pl.ds(sid * RPT + b * IB, IB)], stage)
      pltpu.async_copy(stage, out_hbm.at[widx.at[b]], wsem).wait()
      return 0
    lax.fori_loop(0, WB - 1, wb, 0)
    pltpu.sync_copy(acc.at[pl.ds(sid * RPT + RPT - IB, IB)], stage)
    pltpu.async_copy(stage, out_hbm.at[widx.at[WB - 1]], wsem).wait()

  return deg


def _prep_body(xp_ref, degv_ref, w_ref, u1_ref, dinv_ref):
  d = degv_ref[...]
  deg = d[:, 0] + d[:, 16] + 1.0
  dinv = (1.0 / jnp.sqrt(deg))[:, None]
  dinv_ref[...] = jnp.broadcast_to(dinv, (RB, 128))
  t = jnp.dot(xp_ref[...], w_ref[...], preferred_element_type=jnp.float32)
  u1_ref[...] = t * dinv


def _tc_prep(xpad, degv, w1p):
  return pl.pallas_call(
      _prep_body,
      grid=(NRB,),
      in_specs=[
          pl.BlockSpec((RB, 128), lambda i: (i, 0)),
          pl.BlockSpec((RB, 128), lambda i: (i, 0)),
          pl.BlockSpec((128, 256), lambda i: (0, 0)),
      ],
      out_specs=[
          pl.BlockSpec((RB, 256), lambda i: (i, 0)),
          pl.BlockSpec((RB, 128), lambda i: (i, 0)),
      ],
      out_shape=[
          jax.ShapeDtypeStruct((NN, 256), jnp.float32),
          jax.ShapeDtypeStruct((NN, 128), jnp.float32),
      ],
  )(xpad, degv, w1p)


def _rowmask(i):
  rio = lax.broadcasted_iota(jnp.int32, (RB, 1), 0) + i * RB
  return rio < N


def _bn_cols(y, st_ref, cv_ref, g_ref, be_ref):
  m = st_ref[0, :] / N
  var = cv_ref[0, :] / N
  sd = jnp.sqrt(var + EPS)
  return jnp.maximum(
      (y - m[None, :]) / sd[None, :] * g_ref[...] + be_ref[...], 0.0)


def _make_cvar(f):
  def body(y_ref, st0_ref, cv_ref):
    i = pl.program_id(0)
    m = st0_ref[0, :] / N
    d = jnp.where(_rowmask(i), y_ref[...] - m[None, :], 0.0)

    @pl.when(i == 0)
    def _():
      cv_ref[...] = jnp.zeros_like(cv_ref)
    cv_ref[0, :] += (d * d).sum(axis=0)

  def run(y, st0):
    return pl.pallas_call(
        body,
        grid=(NRB,),
        in_specs=[
            pl.BlockSpec((RB, f), lambda i: (i, 0)),
            pl.BlockSpec((8, f), lambda i: (0, 0)),
        ],
        out_specs=pl.BlockSpec((8, f), lambda i: (0, 0)),
        out_shape=jax.ShapeDtypeStruct((8, f), jnp.float32),
    )(y, st0)
  return run


def _make_bnmm(fin, fout):
  def body(y_ref, st_ref, cv_ref, dinv_ref, w_ref, g_ref, be_ref, u_ref):
    h = _bn_cols(y_ref[...], st_ref, cv_ref, g_ref, be_ref)
    t = jnp.dot(h, w_ref[...], preferred_element_type=jnp.float32)
    t = t * dinv_ref[...][:, :fout]
    if fout < 128:
      u_ref[:, :fout] = t
      u_ref[:, fout:] = jnp.zeros((RB, 128 - fout), jnp.float32)
    else:
      u_ref[...] = t

  def run(y, st, cv, dinv, w, g, be):
    return pl.pallas_call(
        body,
        grid=(NRB,),
        in_specs=[
            pl.BlockSpec((RB, fin), lambda i: (i, 0)),
            pl.BlockSpec((8, fin), lambda i: (0, 0)),
            pl.BlockSpec((8, fin), lambda i: (0, 0)),
            pl.BlockSpec((RB, 128), lambda i: (i, 0)),
            pl.BlockSpec((fin, fout), lambda i: (0, 0)),
            pl.BlockSpec((1, fin), lambda i: (0, 0)),
            pl.BlockSpec((1, fin), lambda i: (0, 0)),
        ],
        out_specs=pl.BlockSpec((RB, 128), lambda i: (i, 0)),
        out_shape=jax.ShapeDtypeStruct((NN, 128), jnp.float32),
    )(y, st, cv, dinv, w, g, be)
  return run


def _make_post(f, sw, uw):
  def body(s_ref, u_ref, dinv_ref, b_ref, y_ref, st_ref):
    i = pl.program_id(0)
    y = ((s_ref[...][:, :f] + u_ref[...][:, :f]) * dinv_ref[...][:, :1]
         + b_ref[...])
    y_ref[...] = y

    @pl.when(i == 0)
    def _():
      st_ref[...] = jnp.zeros_like(st_ref)
    ym = jnp.where(_rowmask(i), y, 0.0)
    st_ref[0, :] += ym.sum(axis=0)
    st_ref[1, :] += (ym * ym).sum(axis=0)

  def run(sv, u, dinv, b):
    return pl.pallas_call(
        body,
        grid=(NRB,),
        in_specs=[
            pl.BlockSpec((RB, sw), lambda i: (i, 0)),
            pl.BlockSpec((RB, uw), lambda i: (i, 0)),
            pl.BlockSpec((RB, 128), lambda i: (i, 0)),
            pl.BlockSpec((1, f), lambda i: (0, 0)),
        ],
        out_specs=[
            pl.BlockSpec((RB, f), lambda i: (i, 0)),
            pl.BlockSpec((8, f), lambda i: (0, 0)),
        ],
        out_shape=[
            jax.ShapeDtypeStruct((NN, f), jnp.float32),
            jax.ShapeDtypeStruct((8, f), jnp.float32),
        ],
    )(sv, u, dinv, b)
  return run


def _pool_body(y_ref, st_ref, cv_ref, g_ref, be_ref, starts_ref, out_ref):
  i = pl.program_id(0)
  h = _bn_cols(y_ref[...], st_ref, cv_ref, g_ref, be_ref)

  @pl.when(i == 0)
  def _():
    out_ref[...] = jnp.zeros_like(out_ref)

  base = i * RB
  rio = lax.broadcasted_iota(jnp.int32, (RB, 1), 0) + base

  def cnt_lo(b, a):
    return a + jnp.where(starts_ref[b + 1] <= base, 1, 0)
  lo = lax.fori_loop(0, BG, cnt_lo, 0)

  def cnt_hi(b, a):
    return a + jnp.where(starts_ref[b] < base + RB, 1, 0)
  hi = lax.fori_loop(0, BG, cnt_hi, 0)

  def seg(b, _):
    s0 = starts_ref[b]
    s1 = starts_ref[b + 1]
    msk = (rio >= s0) & (rio < s1)
    hm = jnp.max(jnp.where(msk, h, 0.0), axis=0)
    cur = out_ref[pl.ds(b, 1), :]
    out_ref[pl.ds(b, 1), :] = jnp.maximum(cur, hm[None, :])
    return 0
  lax.fori_loop(lo, hi, seg, 0)


def _tc_pool(y4, st4, cv4, g, be, starts):
  return pl.pallas_call(
      _pool_body,
      grid=(NRB,),
      in_specs=[
          pl.BlockSpec((RB, 32), lambda i: (i, 0)),
          pl.BlockSpec((8, 32), lambda i: (0, 0)),
          pl.BlockSpec((8, 32), lambda i: (0, 0)),
          pl.BlockSpec((1, 32), lambda i: (0, 0)),
          pl.BlockSpec((1, 32), lambda i: (0, 0)),
          pl.BlockSpec(memory_space=pltpu.SMEM),
      ],
      out_specs=pl.BlockSpec((BG, 32), lambda i: (0, 0)),
      out_shape=jax.ShapeDtypeStruct((BG, 32), jnp.float32),
  )(y4, st4, cv4, g, be, starts)


def _bn_rows(z, g, be):
  m = jnp.mean(z, axis=0)
  zc = z - m[None, :]
  var = jnp.mean(zc * zc, axis=0)
  return zc / jnp.sqrt(var + EPS)[None, :] * g + be


def _head_body(pool_ref, wf1, bf1, gf1, bef1, wf2, bf2, gf2, bef2,
               wf3, bf3, gf3, bef3, wo, bo, out_ref, llf_ref):
  z = jnp.dot(pool_ref[...], wf1[...],
              preferred_element_type=jnp.float32) + bf1[...]
  z = jnp.maximum(_bn_rows(z, gf1[...], bef1[...]), 0.0)
  z = jnp.dot(z, wf2[...], preferred_element_type=jnp.float32) + bf2[...]
  z = jnp.maximum(_bn_rows(z, gf2[...], bef2[...]), 0.0)
  z = jnp.dot(z, wf3[...], preferred_element_type=jnp.float32) + bf3[...]
  llf = jnp.maximum(_bn_rows(z, gf3[...], bef3[...]), 0.0)
  llf_ref[...] = llf
  t = jnp.dot(llf, wo[...], preferred_element_type=jnp.float32) + bo[...]
  out_ref[...] = 1.0 / (1.0 + jnp.exp(-t))


def _tc_head(pool, p):
  r = lambda a: a.reshape(1, -1)
  args = (pool,
          p['Wf1'], r(p['bf1']), r(p['gf1']), r(p['bef1']),
          p['Wf2'], r(p['bf2']), r(p['gf2']), r(p['bef2']),
          p['Wf3'], r(p['bf3']), r(p['gf3']), r(p['bef3']),
          p['Wo'], r(p['bo']))
  return pl.pallas_call(
      _head_body,
      out_shape=[
          jax.ShapeDtypeStruct((BG, 1), jnp.float32),
          jax.ShapeDtypeStruct((BG, 256), jnp.float32),
      ],
  )(*args)


def kernel(x, edge_index, batch, params):
  p = params
  src = edge_index[0]
  dst = edge_index[1]
  e = src.shape[0]

  nb = -(-e // (NTILE * IB))
  nb = -(-nb // GBK) * GBK
  padn = NTILE * nb * IB - e
  pidx = jnp.arange(padn, dtype=jnp.int32)
  src_p = jnp.concatenate([src, (pidx * 97) % N]).reshape(NTILE, nb, IB)
  dst_p = jnp.concatenate([dst, NN + (pidx % 512)]).reshape(NTILE, nb, IB)
  cc = jnp.arange(8, dtype=jnp.int32)
  gidx = src_p[None] * 8 + cc[:, None, None, None]
  cc16 = jnp.arange(16, dtype=jnp.int32)
  gidx16 = src_p[None] * 16 + cc16[:, None, None, None]

  nbd = -(-e // (NWORK * IB))
  nbd = -(-nbd // GBK) * GBK
  padd = NWORK * nbd * IB - e
  pidx2 = jnp.arange(padd, dtype=jnp.int32)
  dst_d = jnp.concatenate([dst, NN + (pidx2 % 512)]).reshape(NWORK, nbd, IB)

  offs = jnp.concatenate([jnp.arange((WB - 1) * IB, dtype=jnp.int32),
                          RPT - IB + jnp.arange(IB, dtype=jnp.int32)])
  trow = (jnp.arange(NTILE, dtype=jnp.int32)[:, None] * RPT + offs[None, :])
  widx = (trow[None] * 8 + cc[:, None, None]).reshape(8, NTILE, WB, IB)
  widx16 = (trow[None] * 16 + cc16[:, None, None]).reshape(16, NTILE, WB, IB)

  starts = jnp.searchsorted(
      batch, jnp.arange(BG + 1, dtype=batch.dtype)).astype(jnp.int32)

  degv = _make_deg(nbd)(dst_d, widx).reshape(NP, 128)

  xpad = jnp.pad(x, ((0, NN - N), (0, 128 - x.shape[1])))
  w1p = jnp.pad(p['W1'], ((0, 128 - p['W1'].shape[0]), (0, 0)))
  u1, dinv = _tc_prep(xpad, degv, w1p)

  prop16 = _make_prop(16, nb, mult=16)
  prop8 = _make_prop(8, nb)
  prop4 = _make_prop(4, nb)
  prop2 = _make_prop(2, nb)
  bnmm1 = _make_bnmm(256, 128)
  bnmm2 = _make_bnmm(128, 64)
  bnmm3 = _make_bnmm(64, 32)
  post1 = _make_post(256, 256, 256)
  post2 = _make_post(128, 128, 128)
  post3 = _make_post(64, 128, 128)
  post4 = _make_post(32, 128, 128)
  r = lambda a: a.reshape(1, -1)

  cvar1 = _make_cvar(256)
  cvar2 = _make_cvar(128)
  cvar3 = _make_cvar(64)
  cvar4 = _make_cvar(32)

  s1 = prop16(u1.reshape(NN * 16, 16), gidx16, dst_p, widx16).reshape(NP, 256)
  y1, st1 = post1(s1, u1, dinv, r(p['b1']))
  cv1 = cvar1(y1, st1)

  u2 = bnmm1(y1, st1, cv1, dinv, p['W2'], r(p['g1']), r(p['be1']))
  s2 = prop8(u2.reshape(NN * 8, 16), gidx, dst_p, widx).reshape(NP, 128)
  y2, st2 = post2(s2, u2, dinv, r(p['b2']))
  cv2 = cvar2(y2, st2)

  u3 = bnmm2(y2, st2, cv2, dinv, p['W3'], r(p['g2']), r(p['be2']))
  s3 = prop4(u3.reshape(NN * 8, 16), gidx, dst_p, widx).reshape(NP, 128)
  y3, st3 = post3(s3, u3, dinv, r(p['b3']))
  cv3 = cvar3(y3, st3)

  u4 = bnmm3(y3, st3, cv3, dinv, p['W4'], r(p['g3']), r(p['be3']))
  s4 = prop2(u4.reshape(NN * 8, 16), gidx, dst_p, widx).reshape(NP, 128)
  y4, st4 = post4(s4, u4, dinv, r(p['b4']))
  cv4 = cvar4(y4, st4)

  pooled = _tc_pool(y4, st4, cv4, r(p['g4']), r(p['be4']), starts)
  out, llf = _tc_head(pooled, p)
  return (out, llf)


# speedup vs baseline: 10.9213x; 1.3429x over previous
import functools

import jax
import jax.numpy as jnp
from jax import lax
from jax.experimental import pallas as pl
from jax.experimental.pallas import tpu as pltpu
from jax.experimental.pallas import tpu_sc as plsc

N = 50000
BG = 256
NSC = 2
NTILE = 16
NWORK = NSC * NTILE
IB = 128
RB = 2048
NN = 51200
NRB = NN // RB
NP = NN + 512
RPT = NP // NTILE
NZ = 4
ZR = RPT // NZ
WB = RPT // IB + 1
EPS = 1e-5
GBK = 4

_scparams = pltpu.CompilerParams(use_tc_tiling_on_sc=False)


def _sc_mesh():
  return plsc.VectorSubcoreMesh(core_axis_name="c", subcore_axis_name="s")


def _make_prop(C, nb, mult=8):
  @functools.partial(
      pl.kernel,
      out_type=jax.ShapeDtypeStruct((NP * mult, 16), jnp.float32),
      mesh=_sc_mesh(),
      compiler_params=_scparams,
      scratch_types=[
          [pltpu.VMEM((GBK, IB), jnp.int32) for _ in range(2)],
          [pltpu.VMEM((GBK, IB), jnp.int32) for _ in range(2)],
          [pltpu.VMEM((IB, 16), jnp.float32) for _ in range(2 * GBK)],
          pltpu.VMEM((ZR, 16), jnp.float32),
          pltpu.VMEM((WB, IB), jnp.int32),
          pltpu.VMEM((IB, 16), jnp.float32),
          pltpu.VMEM_SHARED((NP, 16), jnp.float32),
          [pltpu.SemaphoreType.DMA for _ in range(2 * GBK)],
          pltpu.SemaphoreType.DMA,
      ],
  )
  def prop(u_hbm, gidx_hbm, didx_hbm, widx_hbm, out_hbm,
           idxg, idxd, rows, zbuf, widx, stage, acc, sems, wsem):
    cid = lax.axis_index("c")
    sid = lax.axis_index("s")

    def zfill(i, _):
      zbuf[i, :] = jnp.zeros((16,), jnp.float32)
      return 0
    lax.fori_loop(0, ZR, zfill, 0)

    for c in range(C):
      my = c % NSC

      @pl.when(cid == my)
      def _():
        for z in range(NZ):
          pltpu.sync_copy(zbuf, acc.at[pl.ds(sid * RPT + z * ZR, ZR)])
        plsc.subcore_barrier()

        ngrp = nb // GBK

        def fire(g, ph):
          pltpu.sync_copy(gidx_hbm.at[c, sid, pl.ds(g * GBK, GBK)], idxg[ph])
          pltpu.sync_copy(didx_hbm.at[sid, pl.ds(g * GBK, GBK)], idxd[ph])
          for j in range(GBK):
            pltpu.async_copy(u_hbm.at[idxg[ph].at[j]], rows[ph * GBK + j],
                             sems[ph * GBK + j])

        def drain(g, ph):
          for j in range(GBK):
            pltpu.make_async_copy(u_hbm.at[idxg[ph].at[j]],
                                  rows[ph * GBK + j],
                                  sems[ph * GBK + j]).wait()
            pltpu.sync_copy(rows[ph * GBK + j], acc.at[idxd[ph].at[j]],
                            add=True)

        fire(0, 0)

        def grp2(i, _):
          g = i * 2

          @pl.when(g + 1 < ngrp)
          def _():
            fire(g + 1, 1)
          drain(g, 0)

          @pl.when(g + 2 < ngrp)
          def _():
            fire(g + 2, 0)

          @pl.when(g + 1 < ngrp)
          def _():
            drain(g + 1, 1)
          return 0
        lax.fori_loop(0, (ngrp + 1) // 2, grp2, 0)
        plsc.subcore_barrier()

        pltpu.sync_copy(widx_hbm.at[c, sid], widx)

        def wb(b, _):
          pltpu.sync_copy(acc.at[pl.ds(sid * RPT + b * IB, IB)], stage)
          pltpu.async_copy(stage, out_hbm.at[widx.at[b]], wsem).wait()
          return 0
        lax.fori_loop(0, WB - 1, wb, 0)
        pltpu.sync_copy(acc.at[pl.ds(sid * RPT + RPT - IB, IB)], stage)
        pltpu.async_copy(stage, out_hbm.at[widx.at[WB - 1]], wsem).wait()

  return prop


def _make_deg(nbd):
  @functools.partial(
      pl.kernel,
      out_type=jax.ShapeDtypeStruct((NP * 8, 16), jnp.float32),
      mesh=_sc_mesh(),
      compiler_params=_scparams,
      scratch_types=[
          pltpu.VMEM((GBK, IB), jnp.int32),
          pltpu.VMEM((IB, 16), jnp.float32),
          pltpu.VMEM((ZR, 16), jnp.float32),
          pltpu.VMEM((WB, IB), jnp.int32),
          pltpu.VMEM((IB, 16), jnp.float32),
          pltpu.VMEM_SHARED((NP, 16), jnp.float32),
          pltpu.SemaphoreType.DMA,
      ],
  )
  def deg(didx_hbm, widx_hbm, out_hbm,
          idxd, ones_v, zbuf, widx, stage, acc, wsem):
    cid = lax.axis_index("c")
    sid = lax.axis_index("s")
    wid = cid * NTILE + sid

    def ofill(i, _):
      ones_v[i, :] = jnp.ones((16,), jnp.float32)
      return 0
    lax.fori_loop(0, IB, ofill, 0)

    def zfill(i, _):
      zbuf[i, :] = jnp.zeros((16,), jnp.float32)
      return 0
    lax.fori_loop(0, ZR, zfill, 0)

    for z in range(NZ):
      pltpu.sync_copy(zbuf, acc.at[pl.ds(sid * RPT + z * ZR, ZR)])
    plsc.subcore_barrier()

    def grp(g, _):
      pltpu.sync_copy(didx_hbm.at[wid, pl.ds(g * GBK, GBK)], idxd)
      for j in range(GBK):
        pltpu.sync_copy(ones_v, acc.at[idxd.at[j]], add=True)
      return 0
    lax.fori_loop(0, nbd // GBK, grp, 0)
    plsc.subcore_barrier()

    pltpu.sync_copy(widx_hbm.at[cid, sid], widx)

    def wb(b, _):
      pltpu.sync_copy(acc.at[pl.ds(sid * RPT + b * IB, IB)], stage)
      pltpu.async_copy(stage, out_hbm.at[widx.at[b]], wsem).wait()
      return 0
    lax.fori_loop(0, WB - 1, wb, 0)
    pltpu.sync_copy(acc.at[pl.ds(sid * RPT + RPT - IB, IB)], stage)
    pltpu.async_copy(stage, out_hbm.at[widx.at[WB - 1]], wsem).wait()

  return deg


def _prep_body(xp_ref, degv_ref, w_ref, u1_ref, dinv_ref):
  d = degv_ref[...]
  deg = d[:, 0] + d[:, 16] + 1.0
  dinv = (1.0 / jnp.sqrt(deg))[:, None]
  dinv_ref[...] = jnp.broadcast_to(dinv, (RB, 128))
  t = jnp.dot(xp_ref[...], w_ref[...], preferred_element_type=jnp.float32)
  u1_ref[...] = t * dinv


def _tc_prep(xpad, degv, w1p):
  return pl.pallas_call(
      _prep_body,
      grid=(NRB,),
      in_specs=[
          pl.BlockSpec((RB, 128), lambda i: (i, 0)),
          pl.BlockSpec((RB, 128), lambda i: (i, 0)),
          pl.BlockSpec((128, 256), lambda i: (0, 0)),
      ],
      out_specs=[
          pl.BlockSpec((RB, 256), lambda i: (i, 0)),
          pl.BlockSpec((RB, 128), lambda i: (i, 0)),
      ],
      out_shape=[
          jax.ShapeDtypeStruct((NN, 256), jnp.float32),
          jax.ShapeDtypeStruct((NN, 128), jnp.float32),
      ],
  )(xpad, degv, w1p)


def _rowmask(i):
  rio = lax.broadcasted_iota(jnp.int32, (RB, 1), 0) + i * RB
  return rio < N


def _bn_cols(y, st_ref, cv_ref, g_ref, be_ref):
  m = st_ref[0, :] / N
  var = cv_ref[0, :] / N
  sd = jnp.sqrt(var + EPS)
  return jnp.maximum(
      (y - m[None, :]) / sd[None, :] * g_ref[...] + be_ref[...], 0.0)


def _make_cvar(f):
  def body(y_ref, st0_ref, cv_ref):
    i = pl.program_id(0)
    m = st0_ref[0, :] / N
    d = jnp.where(_rowmask(i), y_ref[...] - m[None, :], 0.0)

    @pl.when(i == 0)
    def _():
      cv_ref[...] = jnp.zeros_like(cv_ref)
    cv_ref[0, :] += (d * d).sum(axis=0)

  def run(y, st0):
    return pl.pallas_call(
        body,
        grid=(NRB,),
        in_specs=[
            pl.BlockSpec((RB, f), lambda i: (i, 0)),
            pl.BlockSpec((8, f), lambda i: (0, 0)),
        ],
        out_specs=pl.BlockSpec((8, f), lambda i: (0, 0)),
        out_shape=jax.ShapeDtypeStruct((8, f), jnp.float32),
    )(y, st0)
  return run


def _make_bnmm(fin, fout):
  def body(y_ref, st_ref, cv_ref, dinv_ref, w_ref, g_ref, be_ref, u_ref):
    h = _bn_cols(y_ref[...], st_ref, cv_ref, g_ref, be_ref)
    t = jnp.dot(h, w_ref[...], preferred_element_type=jnp.float32)
    t = t * dinv_ref[...][:, :fout]
    if fout < 128:
      u_ref[:, :fout] = t
      u_ref[:, fout:] = jnp.zeros((RB, 128 - fout), jnp.float32)
    else:
      u_ref[...] = t

  def run(y, st, cv, dinv, w, g, be):
    return pl.pallas_call(
        body,
        grid=(NRB,),
        in_specs=[
            pl.BlockSpec((RB, fin), lambda i: (i, 0)),
            pl.BlockSpec((8, fin), lambda i: (0, 0)),
            pl.BlockSpec((8, fin), lambda i: (0, 0)),
            pl.BlockSpec((RB, 128), lambda i: (i, 0)),
            pl.BlockSpec((fin, fout), lambda i: (0, 0)),
            pl.BlockSpec((1, fin), lambda i: (0, 0)),
            pl.BlockSpec((1, fin), lambda i: (0, 0)),
        ],
        out_specs=pl.BlockSpec((RB, 128), lambda i: (i, 0)),
        out_shape=jax.ShapeDtypeStruct((NN, 128), jnp.float32),
    )(y, st, cv, dinv, w, g, be)
  return run


def _make_post(f, sw, uw):
  def body(s_ref, u_ref, dinv_ref, b_ref, y_ref, st_ref):
    i = pl.program_id(0)
    y = ((s_ref[...][:, :f] + u_ref[...][:, :f]) * dinv_ref[...][:, :1]
         + b_ref[...])
    y_ref[...] = y

    @pl.when(i == 0)
    def _():
      st_ref[...] = jnp.zeros_like(st_ref)
    ym = jnp.where(_rowmask(i), y, 0.0)
    st_ref[0, :] += ym.sum(axis=0)
    st_ref[1, :] += (ym * ym).sum(axis=0)

  def run(sv, u, dinv, b):
    return pl.pallas_call(
        body,
        grid=(NRB,),
        in_specs=[
            pl.BlockSpec((RB, sw), lambda i: (i, 0)),
            pl.BlockSpec((RB, uw), lambda i: (i, 0)),
            pl.BlockSpec((RB, 128), lambda i: (i, 0)),
            pl.BlockSpec((1, f), lambda i: (0, 0)),
        ],
        out_specs=[
            pl.BlockSpec((RB, f), lambda i: (i, 0)),
            pl.BlockSpec((8, f), lambda i: (0, 0)),
        ],
        out_shape=[
            jax.ShapeDtypeStruct((NN, f), jnp.float32),
            jax.ShapeDtypeStruct((8, f), jnp.float32),
        ],
    )(sv, u, dinv, b)
  return run


def _pool_body(y_ref, st_ref, cv_ref, g_ref, be_ref, starts_ref, out_ref):
  i = pl.program_id(0)
  h = _bn_cols(y_ref[...], st_ref, cv_ref, g_ref, be_ref)

  @pl.when(i == 0)
  def _():
    out_ref[...] = jnp.zeros_like(out_ref)

  base = i * RB
  rio = lax.broadcasted_iota(jnp.int32, (RB, 1), 0) + base

  def cnt_lo(b, a):
    return a + jnp.where(starts_ref[b + 1] <= base, 1, 0)
  lo = lax.fori_loop(0, BG, cnt_lo, 0)

  def cnt_hi(b, a):
    return a + jnp.where(starts_ref[b] < base + RB, 1, 0)
  hi = lax.fori_loop(0, BG, cnt_hi, 0)

  def seg(b, _):
    s0 = starts_ref[b]
    s1 = starts_ref[b + 1]
    msk = (rio >= s0) & (rio < s1)
    hm = jnp.max(jnp.where(msk, h, 0.0), axis=0)
    cur = out_ref[pl.ds(b, 1), :]
    out_ref[pl.ds(b, 1), :] = jnp.maximum(cur, hm[None, :])
    return 0
  lax.fori_loop(lo, hi, seg, 0)


def _tc_pool(y4, st4, cv4, g, be, starts):
  return pl.pallas_call(
      _pool_body,
      grid=(NRB,),
      in_specs=[
          pl.BlockSpec((RB, 32), lambda i: (i, 0)),
          pl.BlockSpec((8, 32), lambda i: (0, 0)),
          pl.BlockSpec((8, 32), lambda i: (0, 0)),
          pl.BlockSpec((1, 32), lambda i: (0, 0)),
          pl.BlockSpec((1, 32), lambda i: (0, 0)),
          pl.BlockSpec(memory_space=pltpu.SMEM),
      ],
      out_specs=pl.BlockSpec((BG, 32), lambda i: (0, 0)),
      out_shape=jax.ShapeDtypeStruct((BG, 32), jnp.float32),
  )(y4, st4, cv4, g, be, starts)


def _bn_rows(z, g, be):
  m = jnp.mean(z, axis=0)
  zc = z - m[None, :]
  var = jnp.mean(zc * zc, axis=0)
  return zc / jnp.sqrt(var + EPS)[None, :] * g + be


def _head_body(pool_ref, wf1, bf1, gf1, bef1, wf2, bf2, gf2, bef2,
               wf3, bf3, gf3, bef3, wo, bo, out_ref, llf_ref):
  z = jnp.dot(pool_ref[...], wf1[...],
              preferred_element_type=jnp.float32) + bf1[...]
  z = jnp.maximum(_bn_rows(z, gf1[...], bef1[...]), 0.0)
  z = jnp.dot(z, wf2[...], preferred_element_type=jnp.float32) + bf2[...]
  z = jnp.maximum(_bn_rows(z, gf2[...], bef2[...]), 0.0)
  z = jnp.dot(z, wf3[...], preferred_element_type=jnp.float32) + bf3[...]
  llf = jnp.maximum(_bn_rows(z, gf3[...], bef3[...]), 0.0)
  llf_ref[...] = llf
  t = jnp.dot(llf, wo[...], preferred_element_type=jnp.float32) + bo[...]
  out_ref[...] = 1.0 / (1.0 + jnp.exp(-t))


def _tc_head(pool, p):
  r = lambda a: a.reshape(1, -1)
  args = (pool,
          p['Wf1'], r(p['bf1']), r(p['gf1']), r(p['bef1']),
          p['Wf2'], r(p['bf2']), r(p['gf2']), r(p['bef2']),
          p['Wf3'], r(p['bf3']), r(p['gf3']), r(p['bef3']),
          p['Wo'], r(p['bo']))
  return pl.pallas_call(
      _head_body,
      out_shape=[
          jax.ShapeDtypeStruct((BG, 1), jnp.float32),
          jax.ShapeDtypeStruct((BG, 256), jnp.float32),
      ],
  )(*args)


def kernel(x, edge_index, batch, params):
  p = params
  src = edge_index[0]
  dst = edge_index[1]
  e = src.shape[0]

  nb = -(-e // (NTILE * IB))
  nb = -(-nb // GBK) * GBK
  padn = NTILE * nb * IB - e
  pidx = jnp.arange(padn, dtype=jnp.int32)
  src_p = jnp.concatenate([src, (pidx * 97) % N]).reshape(NTILE, nb, IB)
  dst_p = jnp.concatenate([dst, NN + (pidx % 512)]).reshape(NTILE, nb, IB)
  cc = jnp.arange(8, dtype=jnp.int32)
  gidx = src_p[None] * 8 + cc[:, None, None, None]
  cc16 = jnp.arange(16, dtype=jnp.int32)
  gidx16 = src_p[None] * 16 + cc16[:, None, None, None]

  nbd = -(-e // (NWORK * IB))
  nbd = -(-nbd // GBK) * GBK
  padd = NWORK * nbd * IB - e
  pidx2 = jnp.arange(padd, dtype=jnp.int32)
  dst_d = jnp.concatenate([dst, NN + (pidx2 % 512)]).reshape(NWORK, nbd, IB)

  offs = jnp.concatenate([jnp.arange((WB - 1) * IB, dtype=jnp.int32),
                          RPT - IB + jnp.arange(IB, dtype=jnp.int32)])
  trow = (jnp.arange(NTILE, dtype=jnp.int32)[:, None] * RPT + offs[None, :])
  widx = (trow[None] * 8 + cc[:, None, None]).reshape(8, NTILE, WB, IB)
  widx16 = (trow[None] * 16 + cc16[:, None, None]).reshape(16, NTILE, WB, IB)

  starts = jnp.searchsorted(
      batch, jnp.arange(BG + 1, dtype=batch.dtype)).astype(jnp.int32)

  degv = _make_deg(nbd)(dst_d, widx).reshape(NP, 128)

  xpad = jnp.pad(x, ((0, NN - N), (0, 128 - x.shape[1])))
  w1p = jnp.pad(p['W1'], ((0, 128 - p['W1'].shape[0]), (0, 0)))
  u1, dinv = _tc_prep(xpad, degv, w1p)

  prop16 = _make_prop(16, nb, mult=16)
  prop8 = _make_prop(8, nb)
  prop4 = _make_prop(4, nb)
  prop2 = _make_prop(2, nb)
  bnmm1 = _make_bnmm(256, 128)
  bnmm2 = _make_bnmm(128, 64)
  bnmm3 = _make_bnmm(64, 32)
  post1 = _make_post(256, 256, 256)
  post2 = _make_post(128, 128, 128)
  post3 = _make_post(64, 128, 128)
  post4 = _make_post(32, 128, 128)
  r = lambda a: a.reshape(1, -1)

  cvar1 = _make_cvar(256)
  cvar2 = _make_cvar(128)
  cvar3 = _make_cvar(64)
  cvar4 = _make_cvar(32)

  s1 = prop16(u1.reshape(NN * 16, 16), gidx16, dst_p, widx16).reshape(NP, 256)
  y1, st1 = post1(s1, u1, dinv, r(p['b1']))
  cv1 = cvar1(y1, st1)

  u2 = bnmm1(y1, st1, cv1, dinv, p['W2'], r(p['g1']), r(p['be1']))
  s2 = prop8(u2.reshape(NN * 8, 16), gidx, dst_p, widx).reshape(NP, 128)
  y2, st2 = post2(s2, u2, dinv, r(p['b2']))
  cv2 = cvar2(y2, st2)

  u3 = bnmm2(y2, st2, cv2, dinv, p['W3'], r(p['g2']), r(p['be2']))
  s3 = prop4(u3.reshape(NN * 8, 16), gidx, dst_p, widx).reshape(NP, 128)
  y3, st3 = post3(s3, u3, dinv, r(p['b3']))
  cv3 = cvar3(y3, st3)

  u4 = bnmm3(y3, st3, cv3, dinv, p['W4'], r(p['g3']), r(p['be3']))
  s4 = prop2(u4.reshape(NN * 8, 16), gidx, dst_p, widx).reshape(NP, 128)
  y4, st4 = post4(s4, u4, dinv, r(p['b4']))
  cv4 = cvar4(y4, st4)

  pooled = _tc_pool(y4, st4, cv4, r(p['g4']), r(p['be4']), starts)
  out, llf = _tc_head(pooled, p)
  return (out, llf)
